# baseline (speedup 1.0000x reference)
import functools

import jax
import jax.numpy as jnp
import numpy as np
from jax import lax
from jax.experimental import pallas as pl
from jax.experimental.pallas import tpu as pltpu
from jax.experimental.pallas import tpu_sc as plsc

L = 16
G = 128


@functools.cache
def _make_sc_bilinear(N, HW, C, W, H):
    info = plsc.get_sparse_core_info()
    NW = info.num_cores * info.num_subcores
    npw = N // NW
    nchunks = npw // G
    mesh = plsc.VectorSubcoreMesh(core_axis_name="c", subcore_axis_name="s")

    @functools.partial(
        pl.kernel,
        mesh=mesh,
        compiler_params=pltpu.CompilerParams(
            use_tc_tiling_on_sc=False, needs_layout_passes=False),
        out_type=jax.ShapeDtypeStruct((N, C), jnp.float32),
        scratch_types=[
            pltpu.VMEM((G,), jnp.float32),
            pltpu.VMEM((G,), jnp.float32),
            pltpu.VMEM((G,), jnp.int32),
            pltpu.VMEM((G,), jnp.int32),
            pltpu.VMEM((G,), jnp.int32),
            pltpu.VMEM((G,), jnp.int32),
            pltpu.VMEM((G,), jnp.float32),
            pltpu.VMEM((G,), jnp.float32),
            pltpu.VMEM((G,), jnp.float32),
            pltpu.VMEM((G,), jnp.float32),
            pltpu.VMEM((G, C), jnp.bfloat16),
            pltpu.VMEM((G, C), jnp.bfloat16),
            pltpu.VMEM((G, C), jnp.bfloat16),
            pltpu.VMEM((G, C), jnp.bfloat16),
            pltpu.VMEM((G, C), jnp.float32),
            pltpu.SemaphoreType.DMA,
        ],
    )
    def sc_kernel(xs_hbm, ys_hbm, table_hbm, out_hbm,
                  xv, yv, i00, i01, i10, i11, w00, w01, w10, w11,
                  r00, r01, r10, r11, ov, sem):
        wid = lax.axis_index("s") * info.num_cores + lax.axis_index("c")
        base0 = wid * npw

        def chunk(ci, carry):
            base = base0 + ci * G
            pltpu.sync_copy(xs_hbm.at[pl.ds(base, G)], xv)
            pltpu.sync_copy(ys_hbm.at[pl.ds(base, G)], yv)

            def grp(j, c2):
                s = pl.ds(j * L, L)
                x = xv[s]
                y = yv[s]
                ix = jnp.clip((x + 1.0) * (0.5 * (W - 1)), 0.0, float(W - 1))
                iy = jnp.clip((y + 1.0) * (0.5 * (H - 1)), 0.0, float(H - 1))
                x0 = ix.astype(jnp.int32)
                y0 = iy.astype(jnp.int32)
                fx = ix - x0.astype(jnp.float32)
                fy = iy - y0.astype(jnp.float32)
                x1 = jnp.minimum(x0 + 1, W - 1)
                y1 = jnp.minimum(y0 + 1, H - 1)
                b0 = y0 * W
                b1 = y1 * W
                i00[s] = b0 + x0
                i01[s] = b0 + x1
                i10[s] = b1 + x0
                i11[s] = b1 + x1
                gx = 1.0 - fx
                gy = 1.0 - fy
                w00[s] = gx * gy
                w01[s] = fx * gy
                w10[s] = gx * fy
                w11[s] = fx * fy
                return c2

            lax.fori_loop(0, G // L, grp, 0)

            cp0 = pltpu.async_copy(table_hbm.at[i00], r00, sem)
            cp1 = pltpu.async_copy(table_hbm.at[i01], r01, sem)
            cp2 = pltpu.async_copy(table_hbm.at[i10], r10, sem)
            cp3 = pltpu.async_copy(table_hbm.at[i11], r11, sem)
            cp0.wait()
            cp1.wait()
            cp2.wait()
            cp3.wait()

            def grp2(j, c2):
                sw = pl.ds(j * L, L)
                a00g = w00[sw]
                a01g = w01[sw]
                a10g = w10[sw]
                a11g = w11[sw]
                for k in range(L):
                    p = j * L + k
                    a00 = a00g[k]
                    a01 = a01g[k]
                    a10 = a10g[k]
                    a11 = a11g[k]
                    for c in range(0, C, 2 * L):
                        s2 = pl.ds(c, 2 * L)
                        e00, o00 = plsc.unpack(
                            r00[p, s2], format=plsc.PackFormat.INTERLEAVED)
                        e01, o01 = plsc.unpack(
                            r01[p, s2], format=plsc.PackFormat.INTERLEAVED)
                        e10, o10 = plsc.unpack(
                            r10[p, s2], format=plsc.PackFormat.INTERLEAVED)
                        e11, o11 = plsc.unpack(
                            r11[p, s2], format=plsc.PackFormat.INTERLEAVED)
                        acc_a = (a00 * e00 + a01 * e01 + a10 * e10 + a11 * e11)
                        acc_b = (a00 * o00 + a01 * o01 + a10 * o10 + a11 * o11)
                        ov[p, pl.ds(c, L)] = acc_a
                        ov[p, pl.ds(c + L, L)] = acc_b
                return c2

            lax.fori_loop(0, G // L, grp2, 0)

            pltpu.sync_copy(ov, out_hbm.at[pl.ds(base, G), :])
            return carry

        lax.fori_loop(0, nchunks, chunk, 0)

    return sc_kernel


def kernel(inp, plane):
    C, H, W = plane.shape
    N = inp.shape[0]
    half = C // 2
    order = np.stack([np.arange(half), half + np.arange(half)], 1).reshape(-1)
    table = (plane.transpose(1, 2, 0).reshape(H * W, C)[:, order]
             .astype(jnp.bfloat16))
    xs = inp[:, 0]
    ys = inp[:, 1]
    return _make_sc_bilinear(N, H * W, C, W, H)(xs, ys, table)


# speedup vs baseline: 1.4795x; 1.4795x over previous
import functools

import jax
import jax.numpy as jnp
from jax import lax
from jax.experimental import pallas as pl
from jax.experimental.pallas import tpu as pltpu
from jax.experimental.pallas import tpu_sc as plsc

L = 16
G = 256


@functools.cache
def _make_sc_bilinear(N, HW, C, W, H):
    info = plsc.get_sparse_core_info()
    NW = info.num_cores * info.num_subcores
    npw = N // NW
    nchunks = npw // G
    CL = C // 4
    mesh = plsc.VectorSubcoreMesh(core_axis_name="c", subcore_axis_name="s")

    @functools.partial(
        pl.kernel,
        mesh=mesh,
        compiler_params=pltpu.CompilerParams(use_tc_tiling_on_sc=False),
        out_type=jax.ShapeDtypeStruct((N // 4, 4 * C), jnp.float32),
        scratch_types=[
            pltpu.VMEM((2, G), jnp.float32),
            pltpu.VMEM((2, G), jnp.float32),
            pltpu.VMEM((2, G), jnp.int32),
            pltpu.VMEM((2, G), jnp.int32),
            pltpu.VMEM((2, G), jnp.int32),
            pltpu.VMEM((2, G), jnp.int32),
            pltpu.VMEM((2, G), jnp.float32),
            pltpu.VMEM((2, G), jnp.float32),
            pltpu.VMEM((2, G), jnp.float32),
            pltpu.VMEM((2, G), jnp.float32),
            pltpu.VMEM((2, G, C), jnp.float32),
            pltpu.VMEM((2, G, C), jnp.float32),
            pltpu.VMEM((2, G, C), jnp.float32),
            pltpu.VMEM((2, G, C), jnp.float32),
            pltpu.VMEM((2, G // 4, 4 * C), jnp.float32),
            pltpu.SemaphoreType.DMA,
            pltpu.SemaphoreType.DMA,
            pltpu.SemaphoreType.DMA,
        ],
    )
    def sc_kernel(xs_hbm, ys_hbm, table_hbm, out_hbm,
                  xv, yv, i00, i01, i10, i11, w00, w01, w10, w11,
                  r00, r01, r10, r11, ov, csem, gsem, osem):
        wid = lax.axis_index("s") * info.num_cores + lax.axis_index("c")
        base0 = wid * npw

        def coords_fetch(ci):
            b = ci % 2
            base = base0 + ci * G
            pltpu.async_copy(xs_hbm.at[pl.ds(base, G)], xv.at[b], csem)
            pltpu.async_copy(ys_hbm.at[pl.ds(base, G)], yv.at[b], csem)

        def coords_wait():
            pltpu.make_async_copy(xs_hbm.at[pl.ds(0, G)], xv.at[0], csem).wait()
            pltpu.make_async_copy(ys_hbm.at[pl.ds(0, G)], yv.at[0], csem).wait()

        def prep(ci):
            b = ci % 2

            def grp(j, c2):
                s = pl.ds(j * L, L)
                x = xv[b, s]
                y = yv[b, s]
                ix = jnp.clip((x + 1.0) * (0.5 * (W - 1)), 0.0, float(W - 1))
                iy = jnp.clip((y + 1.0) * (0.5 * (H - 1)), 0.0, float(H - 1))
                x0 = ix.astype(jnp.int32)
                y0 = iy.astype(jnp.int32)
                fx = ix - x0.astype(jnp.float32)
                fy = iy - y0.astype(jnp.float32)
                x1 = jnp.minimum(x0 + 1, W - 1)
                y1 = jnp.minimum(y0 + 1, H - 1)
                b0 = y0 * W
                b1 = y1 * W
                i00[b, s] = b0 + x0
                i01[b, s] = b0 + x1
                i10[b, s] = b1 + x0
                i11[b, s] = b1 + x1
                gx = 1.0 - fx
                gy = 1.0 - fy
                w00[b, s] = gx * gy
                w01[b, s] = fx * gy
                w10[b, s] = gx * fy
                w11[b, s] = fx * fy
                return c2

            lax.fori_loop(0, G // L, grp, 0)
            pltpu.async_copy(table_hbm.at[i00.at[b]], r00.at[b], gsem)
            pltpu.async_copy(table_hbm.at[i01.at[b]], r01.at[b], gsem)
            pltpu.async_copy(table_hbm.at[i10.at[b]], r10.at[b], gsem)
            pltpu.async_copy(table_hbm.at[i11.at[b]], r11.at[b], gsem)

        def gather_wait(b):
            pltpu.make_async_copy(table_hbm.at[i00.at[b]], r00.at[b], gsem).wait()
            pltpu.make_async_copy(table_hbm.at[i01.at[b]], r01.at[b], gsem).wait()
            pltpu.make_async_copy(table_hbm.at[i10.at[b]], r10.at[b], gsem).wait()
            pltpu.make_async_copy(table_hbm.at[i11.at[b]], r11.at[b], gsem).wait()

        def blend(ci):
            b = ci % 2
            base = base0 + ci * G
            gather_wait(b)

            def grp2(j, c2):
                sw = pl.ds(j * L, L)
                a00g = w00[b, sw]
                a01g = w01[b, sw]
                a10g = w10[b, sw]
                a11g = w11[b, sw]
                for k in range(L):
                    p = j * L + k
                    a00 = a00g[k]
                    a01 = a01g[k]
                    a10 = a10g[k]
                    a11 = a11g[k]
                    orow = 4 * j + k // 4
                    ocol = (k % 4) * C
                    for c in range(0, C, L):
                        s = pl.ds(c, L)
                        acc = (a00 * r00[b, p, s] + a01 * r01[b, p, s]
                               + a10 * r10[b, p, s] + a11 * r11[b, p, s])
                        ov[b, orow, pl.ds(ocol + c, L)] = acc
                return c2

            lax.fori_loop(0, G // L, grp2, 0)
            pltpu.async_copy(
                ov.at[b], out_hbm.at[pl.ds((base // 4), G // 4), :], osem)

        def out_drain(b):
            pltpu.make_async_copy(
                ov.at[b], out_hbm.at[pl.ds(0, G // 4), :], osem).wait()

        coords_fetch(0)
        coords_fetch(1)
        coords_wait()
        prep(0)

        def body(i, carry):
            @pl.when(i + 2 < nchunks)
            def _():
                coords_fetch(i + 2)

            @pl.when(i + 1 < nchunks)
            def _():
                coords_wait()
                prep(i + 1)

            @pl.when(i >= 2)
            def _():
                out_drain(i % 2)

            blend(i)
            return carry

        lax.fori_loop(0, nchunks, body, 0)
        out_drain(nchunks % 2)
        out_drain((nchunks + 1) % 2)

    return sc_kernel


def kernel(inp, plane):
    C, H, W = plane.shape
    N = inp.shape[0]
    table = plane.transpose(1, 2, 0).reshape(H * W, C)
    xs = inp[:, 0]
    ys = inp[:, 1]
    out4 = _make_sc_bilinear(N, H * W, C, W, H)(xs, ys, table)
    return out4.reshape(N, C)
